# Initial kernel scaffold; baseline (speedup 1.0000x reference)
import jax
import jax.numpy as jnp
from jax.experimental import pallas as pl


def kernel(tokens_1gram, tokens_2gram, tokens_3gram, edge_index, emb1, emb2, emb3, W_gcn, b_gcn, W_av, b_av, W_ac, b_ac, W_c, b_c, W_i, b_i):
    raise NotImplementedError("write your pallas kernel here")


# speedup vs baseline: 2.0909x; 2.0909x over previous
import functools

import jax
import jax.numpy as jnp
import numpy as np
from jax import lax
from jax.experimental import pallas as pl
from jax.experimental.pallas import tpu as pltpu
from jax.experimental.pallas import tpu_sc as plsc

N_NODES_C = 10000
TOK_LEN_C = 32
V_C = 20000
D_IN_C = 1024
D_OUT_C = 300

NC = 2
NS = 16
NW = NC * NS
WIN = 128

SEC1 = 10240
DEST1 = 2 * SEC1
DEST3 = 10240
DEST5 = 60448

T1 = 327680
T3 = 163840
T5 = 983040


def _sc_gather_scatter_add(nwin: int, dest: int):
  mesh = plsc.VectorSubcoreMesh(core_axis_name="c", subcore_axis_name="s")

  @functools.partial(
      pl.kernel,
      out_type=jax.ShapeDtypeStruct((NC, dest), jnp.float32),
      mesh=mesh,
      scratch_types=[
          pltpu.VMEM((WIN,), jnp.int32),
          pltpu.VMEM((WIN,), jnp.int32),
          pltpu.VMEM((WIN,), jnp.float32),
          pltpu.VMEM_SHARED((dest,), jnp.float32),
      ],
  )
  def k(vals_hbm, gidx_hbm, sidx_hbm, zeros_hbm, out_hbm, gbuf, sbuf, vbuf, acc):
    cid = lax.axis_index("c")
    sid = lax.axis_index("s")
    wid = sid * NC + cid

    @pl.when(sid == 0)
    def _():
      pltpu.sync_copy(zeros_hbm, acc)

    plsc.subcore_barrier()

    def body(w, carry):
      base = (wid * nwin + w) * WIN
      pltpu.sync_copy(gidx_hbm.at[pl.ds(base, WIN)], gbuf)
      pltpu.sync_copy(sidx_hbm.at[pl.ds(base, WIN)], sbuf)
      pltpu.sync_copy(vals_hbm.at[gbuf], vbuf)
      pltpu.sync_copy(vbuf, acc.at[sbuf], add=True)
      return carry

    lax.fori_loop(0, nwin, body, 0)
    plsc.subcore_barrier()

    @pl.when(sid == 0)
    def _():
      pltpu.sync_copy(acc, out_hbm.at[cid])

  return k


_sc_deg = _sc_gather_scatter_add(T1 // (NW * WIN), DEST1)
_sc_agg = _sc_gather_scatter_add(T3 // (NW * WIN), DEST3)
_sc_tok = _sc_gather_scatter_add(T5 // (NW * WIN), DEST5)


def _norm_body(d_ref, o_ref):
  deg = d_ref[0, :] + d_ref[1, :]
  o_ref[...] = lax.rsqrt(jnp.maximum(deg, 1.0))


def _c_body(n_ref, t_ref, o_ref):
  o_ref[...] = n_ref[...] * (t_ref[0, :] + t_ref[1, :])


_KC = 1000


def _final_body(w1_ref, w2_ref, w3_ref, e1_ref, e2_ref, e3_ref,
                wg_ref, bg_ref, wav_ref, bav_ref, wac_ref, bac_ref,
                wc_ref, bc_ref, wi_ref, bi_ref,
                oav_ref, oac_ref, oc_ref, oi_ref, acc_ref):
  kstep = pl.program_id(0)

  @pl.when(kstep == 0)
  def _():
    acc_ref[...] = jnp.zeros_like(acc_ref)

  part = jnp.zeros((1, D_IN_C), jnp.float32)
  for w_ref, e_ref in ((w1_ref, e1_ref), (w2_ref, e2_ref), (w3_ref, e3_ref)):
    w = (w_ref[0, 0, 0, :] + w_ref[0, 0, 1, :]) * (1.0 / 96.0)
    part = part + jnp.dot(w[None, :], e_ref[0], precision=lax.Precision.HIGHEST,
                          preferred_element_type=jnp.float32)
  acc_ref[...] += part

  @pl.when(kstep == pl.num_programs(0) - 1)
  def _():
    s_vec = acc_ref[...]
    hg = jnp.dot(s_vec, wg_ref[...], precision=lax.Precision.HIGHEST,
                 preferred_element_type=jnp.float32)
    hg = hg * (1.0 / N_NODES_C) + bg_ref[...]
    hg = jnp.maximum(hg, 0.0)
    oav_ref[...] = jnp.dot(hg, wav_ref[...], precision=lax.Precision.HIGHEST, preferred_element_type=jnp.float32) + bav_ref[...]
    oac_ref[...] = jnp.dot(hg, wac_ref[...], precision=lax.Precision.HIGHEST, preferred_element_type=jnp.float32) + bac_ref[...]
    oc_ref[...] = jnp.dot(hg, wc_ref[...], precision=lax.Precision.HIGHEST, preferred_element_type=jnp.float32) + bc_ref[...]
    oi_ref[...] = jnp.dot(hg, wi_ref[...], precision=lax.Precision.HIGHEST, preferred_element_type=jnp.float32) + bi_ref[...]


def _final_call(w_p, emb1, emb2, emb3, w_gcn, b_gcn,
                w_av, b_av, w_ac, b_ac, w_c, b_c, w_i, b_i):
  nk = V_C // _KC
  w3d = w_p[:, :3 * V_C].reshape(2, 3, nk, _KC).transpose(1, 2, 0, 3)
  emb1 = emb1.reshape(nk, _KC, D_IN_C)
  emb2 = emb2.reshape(nk, _KC, D_IN_C)
  emb3 = emb3.reshape(nk, _KC, D_IN_C)
  w_spec = lambda g: pl.BlockSpec((1, 1, 2, _KC), lambda k, g=g: (g, k, 0, 0))
  e_spec = pl.BlockSpec((1, _KC, D_IN_C), lambda k: (k, 0, 0))
  full = lambda shp: pl.BlockSpec(shp, lambda k: (0,) * len(shp))
  return pl.pallas_call(
      _final_body,
      grid=(nk,),
      in_specs=[
          w_spec(0), w_spec(1), w_spec(2),
          e_spec, e_spec, e_spec,
          full((D_IN_C, D_OUT_C)), full((1, D_OUT_C)),
          full((D_OUT_C, 4)), full((1, 4)),
          full((D_OUT_C, 2)), full((1, 2)),
          full((D_OUT_C, 3)), full((1, 3)),
          full((D_OUT_C, 3)), full((1, 3)),
      ],
      out_specs=[full((1, 4)), full((1, 2)), full((1, 3)), full((1, 3))],
      out_shape=[
          jax.ShapeDtypeStruct((1, 4), jnp.float32),
          jax.ShapeDtypeStruct((1, 2), jnp.float32),
          jax.ShapeDtypeStruct((1, 3), jnp.float32),
          jax.ShapeDtypeStruct((1, 3), jnp.float32),
      ],
      scratch_shapes=[pltpu.VMEM((1, D_IN_C), jnp.float32)],
  )(w3d, w3d, w3d, emb1, emb2, emb3, w_gcn, b_gcn,
    w_av, b_av, w_ac, b_ac, w_c, b_c, w_i, b_i)


_GIDX1 = np.arange(T1, dtype=np.int32) % 128
_PAD1A = 10000 + np.arange(3840, dtype=np.int32) % 240
_PAD1B = SEC1 + 10000 + np.arange(3840, dtype=np.int32) % 240
_PAD3G = SEC1 + np.arange(3840, dtype=np.int32) % 10000
_PAD3S = 10000 + np.arange(3840, dtype=np.int32) % 240
_NODE_IDS = np.concatenate([
    np.tile(np.repeat(np.arange(N_NODES_C, dtype=np.int32), TOK_LEN_C), 3),
    np.arange(T5 - 3 * N_NODES_C * TOK_LEN_C, dtype=np.int32) % 10000,
])
_PAD5S = (60000 + np.arange(T5 - 3 * N_NODES_C * TOK_LEN_C,
                            dtype=np.int32) % 448)
_ONES128 = np.ones((128,), np.float32)
_Z1 = np.zeros((DEST1,), np.float32)
_Z3 = np.zeros((DEST3,), np.float32)
_Z5 = np.zeros((DEST5,), np.float32)


def kernel(tokens_1gram, tokens_2gram, tokens_3gram, edge_index,
           emb1, emb2, emb3, W_gcn, b_gcn,
           W_av, b_av, W_ac, b_ac, W_c, b_c, W_i, b_i):
  src = edge_index[0].astype(jnp.int32)
  dst = edge_index[1].astype(jnp.int32)

  sidx1 = jnp.concatenate([src, _PAD1A, dst + SEC1, _PAD1B])
  deg_p = _sc_deg(_ONES128, _GIDX1, sidx1, _Z1)

  norm_both = pl.pallas_call(
      _norm_body,
      out_shape=jax.ShapeDtypeStruct((DEST1,), jnp.float32),
  )(deg_p)

  gidx3 = jnp.concatenate([dst + SEC1, _PAD3G])
  sidx3 = jnp.concatenate([src, _PAD3S])
  t_p = _sc_agg(norm_both, gidx3, sidx3, _Z3)

  c_full = pl.pallas_call(
      _c_body,
      out_shape=jax.ShapeDtypeStruct((DEST3,), jnp.float32),
  )(norm_both[:SEC1], t_p)

  tok = jnp.concatenate([
      tokens_1gram.astype(jnp.int32).ravel(),
      tokens_2gram.astype(jnp.int32).ravel() + V_C,
      tokens_3gram.astype(jnp.int32).ravel() + 2 * V_C,
      _PAD5S,
  ])
  w_p = _sc_tok(c_full, _NODE_IDS, tok, _Z5)

  return tuple(_final_call(
      w_p, emb1, emb2, emb3, W_gcn, b_gcn.reshape(1, -1),
      W_av, b_av.reshape(1, -1), W_ac, b_ac.reshape(1, -1),
      W_c, b_c.reshape(1, -1), W_i, b_i.reshape(1, -1)))


# speedup vs baseline: 10.7337x; 5.1335x over previous
import functools

import jax
import jax.numpy as jnp
import numpy as np
from jax import lax
from jax.experimental import pallas as pl
from jax.experimental.pallas import tpu as pltpu
from jax.experimental.pallas import tpu_sc as plsc

N_NODES_C = 10000
TOK_LEN_C = 32
V_C = 20000
D_IN_C = 1024
D_OUT_C = 300

NC = 2
NS = 16
NW = NC * NS
WIN = 128

SEC1 = 10240
DEST1 = 2 * SEC1
DEST3 = 10240
DEST5 = 60448

T1 = 327680
T3 = 163840
T5 = 983040


def _sc_gather_scatter_add(nwin: int, sb: int, dest: int, has_gather: bool):
  mesh = plsc.VectorSubcoreMesh(core_axis_name="c", subcore_axis_name="s")
  nsup = nwin // sb
  scratch = [
      pltpu.VMEM((sb, 1, WIN), jnp.int32),
      pltpu.VMEM((sb, 1, WIN), jnp.float32),
      pltpu.VMEM_SHARED((dest,), jnp.float32),
      pltpu.SemaphoreType.DMA,
  ]
  if has_gather:
    scratch.append(pltpu.VMEM((sb, 1, WIN), jnp.int32))
    scratch.append(pltpu.SemaphoreType.DMA)

  @functools.partial(
      pl.kernel,
      out_type=jax.ShapeDtypeStruct((NC, dest), jnp.float32),
      mesh=mesh,
      scratch_types=scratch,
  )
  def k(vals_hbm, gidx_hbm, sidx_hbm, zeros_hbm, out_hbm, sbuf, vbuf, acc,
        ssem, *rest):
    cid = lax.axis_index("c")
    sid = lax.axis_index("s")
    wid = sid * NC + cid

    @pl.when(sid == 0)
    def _():
      pltpu.sync_copy(zeros_hbm, acc)

    if has_gather:
      gbuf, gsem = rest
    else:
      for j in range(sb):
        def fill(i, carry, j=j):
          vbuf[j, 0, pl.ds(i * 16, 16)] = jnp.full((16,), 1.0, jnp.float32)
          return carry
        lax.fori_loop(0, WIN // 16, fill, 0)

    plsc.subcore_barrier()

    def body(s, carry):
      base = (wid * nsup + s) * sb
      pltpu.sync_copy(sidx_hbm.at[pl.ds(base, sb)], sbuf)
      if has_gather:
        pltpu.sync_copy(gidx_hbm.at[pl.ds(base, sb)], gbuf)
        gd = [
            pltpu.async_copy(vals_hbm.at[gbuf.at[j, 0]], vbuf.at[j, 0], gsem)
            for j in range(sb)
        ]
        for d in gd:
          d.wait()
      sd = [
          pltpu.async_copy(vbuf.at[j, 0], acc.at[sbuf.at[j, 0]], ssem,
                           add=True)
          for j in range(sb)
      ]
      for d in sd:
        d.wait()
      return carry

    lax.fori_loop(0, nsup, body, 0)
    plsc.subcore_barrier()

    @pl.when(sid == 0)
    def _():
      pltpu.sync_copy(acc, out_hbm.at[cid])

  return k


_sc_deg = _sc_gather_scatter_add(T1 // (NW * WIN), 8, DEST1, False)
_sc_agg = _sc_gather_scatter_add(T3 // (NW * WIN), 8, DEST3, True)
_sc_tok = _sc_gather_scatter_add(T5 // (NW * WIN), 8, DEST5, True)


def _norm_body(d_ref, o_ref):
  deg = d_ref[0, :] + d_ref[1, :]
  o_ref[...] = lax.rsqrt(jnp.maximum(deg, 1.0))


def _c_body(n_ref, t_ref, o_ref):
  o_ref[...] = n_ref[...] * (t_ref[0, :] + t_ref[1, :])


_KC = 1000


def _final_body(w1_ref, w2_ref, w3_ref, e1_ref, e2_ref, e3_ref,
                wg_ref, bg_ref, wav_ref, bav_ref, wac_ref, bac_ref,
                wc_ref, bc_ref, wi_ref, bi_ref,
                oav_ref, oac_ref, oc_ref, oi_ref, acc_ref):
  kstep = pl.program_id(0)

  @pl.when(kstep == 0)
  def _():
    acc_ref[...] = jnp.zeros_like(acc_ref)

  part = jnp.zeros((1, D_IN_C), jnp.float32)
  for w_ref, e_ref in ((w1_ref, e1_ref), (w2_ref, e2_ref), (w3_ref, e3_ref)):
    w = (w_ref[0, 0, 0, :] + w_ref[0, 0, 1, :]) * (1.0 / 96.0)
    part = part + jnp.dot(w[None, :], e_ref[0], precision=lax.Precision.HIGHEST,
                          preferred_element_type=jnp.float32)
  acc_ref[...] += part

  @pl.when(kstep == pl.num_programs(0) - 1)
  def _():
    s_vec = acc_ref[...]
    hg = jnp.dot(s_vec, wg_ref[...], precision=lax.Precision.HIGHEST,
                 preferred_element_type=jnp.float32)
    hg = hg * (1.0 / N_NODES_C) + bg_ref[...]
    hg = jnp.maximum(hg, 0.0)
    oav_ref[...] = jnp.dot(hg, wav_ref[...], precision=lax.Precision.HIGHEST, preferred_element_type=jnp.float32) + bav_ref[...]
    oac_ref[...] = jnp.dot(hg, wac_ref[...], precision=lax.Precision.HIGHEST, preferred_element_type=jnp.float32) + bac_ref[...]
    oc_ref[...] = jnp.dot(hg, wc_ref[...], precision=lax.Precision.HIGHEST, preferred_element_type=jnp.float32) + bc_ref[...]
    oi_ref[...] = jnp.dot(hg, wi_ref[...], precision=lax.Precision.HIGHEST, preferred_element_type=jnp.float32) + bi_ref[...]


def _final_call(w_p, emb1, emb2, emb3, w_gcn, b_gcn,
                w_av, b_av, w_ac, b_ac, w_c, b_c, w_i, b_i):
  nk = V_C // _KC
  w3d = w_p[:, :3 * V_C].reshape(2, 3, nk, _KC).transpose(1, 2, 0, 3)
  emb1 = emb1.reshape(nk, _KC, D_IN_C)
  emb2 = emb2.reshape(nk, _KC, D_IN_C)
  emb3 = emb3.reshape(nk, _KC, D_IN_C)
  w_spec = lambda g: pl.BlockSpec((1, 1, 2, _KC), lambda k, g=g: (g, k, 0, 0))
  e_spec = pl.BlockSpec((1, _KC, D_IN_C), lambda k: (k, 0, 0))
  full = lambda shp: pl.BlockSpec(shp, lambda k: (0,) * len(shp))
  return pl.pallas_call(
      _final_body,
      grid=(nk,),
      in_specs=[
          w_spec(0), w_spec(1), w_spec(2),
          e_spec, e_spec, e_spec,
          full((D_IN_C, D_OUT_C)), full((1, D_OUT_C)),
          full((D_OUT_C, 4)), full((1, 4)),
          full((D_OUT_C, 2)), full((1, 2)),
          full((D_OUT_C, 3)), full((1, 3)),
          full((D_OUT_C, 3)), full((1, 3)),
      ],
      out_specs=[full((1, 4)), full((1, 2)), full((1, 3)), full((1, 3))],
      out_shape=[
          jax.ShapeDtypeStruct((1, 4), jnp.float32),
          jax.ShapeDtypeStruct((1, 2), jnp.float32),
          jax.ShapeDtypeStruct((1, 3), jnp.float32),
          jax.ShapeDtypeStruct((1, 3), jnp.float32),
      ],
      scratch_shapes=[pltpu.VMEM((1, D_IN_C), jnp.float32)],
  )(w3d, w3d, w3d, emb1, emb2, emb3, w_gcn, b_gcn,
    w_av, b_av, w_ac, b_ac, w_c, b_c, w_i, b_i)


_GIDX1 = np.arange(T1, dtype=np.int32) % 128
_PAD1A = 10000 + np.arange(3840, dtype=np.int32) % 240
_PAD1B = SEC1 + 10000 + np.arange(3840, dtype=np.int32) % 240
_PAD3G = SEC1 + np.arange(3840, dtype=np.int32) % 10000
_PAD3S = 10000 + np.arange(3840, dtype=np.int32) % 240
_NODE_IDS = np.concatenate([
    np.tile(np.repeat(np.arange(N_NODES_C, dtype=np.int32), TOK_LEN_C), 3),
    np.arange(T5 - 3 * N_NODES_C * TOK_LEN_C, dtype=np.int32) % 10000,
])
_PAD5S = (60000 + np.arange(T5 - 3 * N_NODES_C * TOK_LEN_C,
                            dtype=np.int32) % 448)
_ONES128 = np.ones((128,), np.float32)
_Z1 = np.zeros((DEST1,), np.float32)
_Z3 = np.zeros((DEST3,), np.float32)
_Z5 = np.zeros((DEST5,), np.float32)


def kernel(tokens_1gram, tokens_2gram, tokens_3gram, edge_index,
           emb1, emb2, emb3, W_gcn, b_gcn,
           W_av, b_av, W_ac, b_ac, W_c, b_c, W_i, b_i):
  src = edge_index[0].astype(jnp.int32)
  dst = edge_index[1].astype(jnp.int32)

  sidx1 = jnp.concatenate([src, _PAD1A, dst + SEC1, _PAD1B])
  deg_p = _sc_deg(_ONES128, _GIDX1[:WIN], sidx1.reshape(T1 // WIN, 1, WIN), _Z1)

  norm_both = pl.pallas_call(
      _norm_body,
      out_shape=jax.ShapeDtypeStruct((DEST1,), jnp.float32),
  )(deg_p)

  gidx3 = jnp.concatenate([dst + SEC1, _PAD3G])
  sidx3 = jnp.concatenate([src, _PAD3S])
  t_p = _sc_agg(norm_both, gidx3.reshape(T3 // WIN, 1, WIN),
                sidx3.reshape(T3 // WIN, 1, WIN), _Z3)

  c_full = pl.pallas_call(
      _c_body,
      out_shape=jax.ShapeDtypeStruct((DEST3,), jnp.float32),
  )(norm_both[:SEC1], t_p)

  tok = jnp.concatenate([
      tokens_1gram.astype(jnp.int32).ravel(),
      tokens_2gram.astype(jnp.int32).ravel() + V_C,
      tokens_3gram.astype(jnp.int32).ravel() + 2 * V_C,
      _PAD5S,
  ])
  w_p = _sc_tok(c_full, _NODE_IDS.reshape(T5 // WIN, 1, WIN),
                tok.reshape(T5 // WIN, 1, WIN), _Z5)

  return tuple(_final_call(
      w_p, emb1, emb2, emb3, W_gcn, b_gcn.reshape(1, -1),
      W_av, b_av.reshape(1, -1), W_ac, b_ac.reshape(1, -1),
      W_c, b_c.reshape(1, -1), W_i, b_i.reshape(1, -1)))
